# Initial kernel scaffold; baseline (speedup 1.0000x reference)
import jax
import jax.numpy as jnp
from jax.experimental import pallas as pl


def kernel(x, edge_index, edge_weight, W1, W2):
    raise NotImplementedError("write your pallas kernel here")


# speedup vs baseline: 3.7093x; 3.7093x over previous
import functools

import jax
import jax.numpy as jnp
from jax import lax
from jax.experimental import pallas as pl
from jax.experimental.pallas import tpu as pltpu
from jax.experimental.pallas import tpu_sc as plsc

_N = 10000
_E = 320000
_D = 256
_H1 = 128
_H2 = 64

_NP = 10240
_NC = 2
_NS = 16
_NW = _NC * _NS
_CHUNK = 80


def _mm_xw1(x, w1):
  blk = 1000

  def body(x_ref, w_ref, o_ref):
    o_ref[...] = jnp.dot(x_ref[...], w_ref[...],
                         preferred_element_type=jnp.float32)

  return pl.pallas_call(
      body,
      grid=(_N // blk,),
      in_specs=[
          pl.BlockSpec((blk, _D), lambda i: (i, 0)),
          pl.BlockSpec((_D, _H1), lambda i: (0, 0)),
      ],
      out_specs=pl.BlockSpec((blk, _H1), lambda i: (i, 0)),
      out_shape=jax.ShapeDtypeStruct((_N, _H1), jnp.float32),
  )(x, w1)


def _relu_add_mm(p, w2p):
  blk = 1000

  def body(p_ref, w_ref, o_ref):
    h = jax.nn.relu(p_ref[0] + p_ref[1])
    o_ref[...] = jnp.dot(h, w_ref[...], preferred_element_type=jnp.float32)

  return pl.pallas_call(
      body,
      grid=(_N // blk,),
      in_specs=[
          pl.BlockSpec((2, blk, _H1), lambda i: (0, i, 0)),
          pl.BlockSpec((_H1, 128), lambda i: (0, 0)),
      ],
      out_specs=pl.BlockSpec((blk, 128), lambda i: (i, 0)),
      out_shape=jax.ShapeDtypeStruct((_N, 128), jnp.float32),
  )(p, w2p)


def _add2(q):
  blk = 1000

  def body(q_ref, o_ref):
    o_ref[...] = (q_ref[0] + q_ref[1])[:, :_H2]

  return pl.pallas_call(
      body,
      grid=(_N // blk,),
      in_specs=[pl.BlockSpec((2, blk, 128), lambda i: (0, i, 0))],
      out_specs=pl.BlockSpec((blk, _H2), lambda i: (i, 0)),
      out_shape=jax.ShapeDtypeStruct((_N, _H2), jnp.float32),
  )(q)


def _zzt(z):
  blk = 400

  def body(zi_ref, zj_ref, o_ref):
    o_ref[...] = lax.dot_general(
        zi_ref[...], zj_ref[...], (((1,), (1,)), ((), ())),
        preferred_element_type=jnp.float32)

  return pl.pallas_call(
      body,
      grid=(_N // blk,),
      in_specs=[
          pl.BlockSpec((blk, _H2), lambda i: (i, 0)),
          pl.BlockSpec((_N, _H2), lambda i: (0, 0)),
      ],
      out_specs=pl.BlockSpec((blk, _N), lambda i: (i, 0)),
      out_shape=jax.ShapeDtypeStruct((_N, _N), jnp.float32),
  )(z, z)


def _make_seg_sum(h_feat, scale_feat):
  edges_per_worker = _E // _NW
  n_chunks = edges_per_worker // _CHUNK
  rows_per_tile = _NP // _NS
  mesh = plsc.VectorSubcoreMesh(
      core_axis_name="c", subcore_axis_name="s",
      num_cores=_NC, num_subcores=_NS)

  @functools.partial(
      pl.kernel,
      out_type=jax.ShapeDtypeStruct((_NC, _NP, h_feat), jnp.float32),
      mesh=mesh,
      scratch_types=[
          pltpu.VMEM((_CHUNK,), jnp.int32),
          pltpu.VMEM((_CHUNK,), jnp.int32),
          pltpu.VMEM((_CHUNK,), jnp.float32),
          pltpu.VMEM((_CHUNK, h_feat), jnp.float32),
          pltpu.VMEM_SHARED((_NP, h_feat), jnp.float32),
          pltpu.SemaphoreType.DMA,
      ],
  )
  def seg(h_hbm, src_hbm, dst_hbm, w_hbm, zeros_hbm, out_hbm,
          src_v, dst_v, w_v, rows_v, acc_sh, sem):
    cid = lax.axis_index("c")
    sid = lax.axis_index("s")
    wid = cid * _NS + sid

    r0 = sid * rows_per_tile
    pltpu.sync_copy(zeros_hbm.at[pl.ds(r0, rows_per_tile)],
                    acc_sh.at[pl.ds(r0, rows_per_tile)])
    plsc.subcore_barrier()

    base = wid * edges_per_worker

    def chunk_body(i, carry):
      off = base + i * _CHUNK
      pltpu.sync_copy(src_hbm.at[pl.ds(off, _CHUNK)], src_v)
      pltpu.sync_copy(dst_hbm.at[pl.ds(off, _CHUNK)], dst_v)
      pltpu.sync_copy(w_hbm.at[pl.ds(off, _CHUNK)], w_v)
      pltpu.async_copy(h_hbm.at[src_v], rows_v, sem).wait()

      def group_body(g, carry2):
        wvec = w_v[pl.ds(g * 16, 16)]
        for l in range(16):
          r = g * 16 + l
          wr = wvec[l]
          for j in range(scale_feat // 16):
            sl = pl.ds(j * 16, 16)
            rows_v[r, sl] = rows_v[r, sl] * wr
        return carry2

      lax.fori_loop(0, _CHUNK // 16, group_body, 0, unroll=False)

      pltpu.sync_copy(rows_v, acc_sh.at[dst_v], add=True)
      return carry

    lax.fori_loop(0, n_chunks, chunk_body, 0, unroll=False)
    plsc.subcore_barrier()

    pltpu.sync_copy(acc_sh.at[pl.ds(r0, rows_per_tile)],
                    out_hbm.at[cid, pl.ds(r0, rows_per_tile)])

  return seg


_seg_sum_h1 = _make_seg_sum(_H1, _H1)
_seg_sum_h2 = _make_seg_sum(128, _H2)


def kernel(x, edge_index, edge_weight, W1, W2):
  src = edge_index[0].astype(jnp.int32)
  dst = edge_index[1].astype(jnp.int32)
  w = edge_weight.astype(jnp.float32)
  zeros1 = jnp.zeros((_NP, _H1), jnp.float32)
  zeros2 = jnp.zeros((_NP, 128), jnp.float32)

  h = _mm_xw1(x, W1)
  p = _seg_sum_h1(h, src, dst, w, zeros1)
  w2p = jnp.pad(W2, ((0, 0), (0, 128 - _H2)))
  h2 = _relu_add_mm(p, w2p)
  q = _seg_sum_h2(h2, src, dst, w, zeros2)
  z = _add2(q)
  adj = _zzt(z)
  return (adj, z)


# speedup vs baseline: 7.7779x; 2.0969x over previous
import functools

import jax
import jax.numpy as jnp
from jax import lax
from jax.experimental import pallas as pl
from jax.experimental.pallas import tpu as pltpu
from jax.experimental.pallas import tpu_sc as plsc

_N = 10000
_E = 320000
_D = 256
_H1 = 128
_H2 = 64

_NP = 10240
_NC = 2
_NS = 16
_NW = _NC * _NS
_CHUNK = 80


def _mm_xw1(x, w1):
  blk = 1000

  def body(x_ref, w_ref, o_ref):
    o_ref[...] = jnp.dot(x_ref[...], w_ref[...],
                         preferred_element_type=jnp.float32)

  return pl.pallas_call(
      body,
      grid=(_N // blk,),
      in_specs=[
          pl.BlockSpec((blk, _D), lambda i: (i, 0)),
          pl.BlockSpec((_D, _H1), lambda i: (0, 0)),
      ],
      out_specs=pl.BlockSpec((blk, _H1), lambda i: (i, 0)),
      out_shape=jax.ShapeDtypeStruct((_N, _H1), jnp.float32),
  )(x, w1)


def _relu_add_mm(p, w2p):
  blk = 1000

  def body(p_ref, w_ref, o_ref):
    h = jax.nn.relu(p_ref[0] + p_ref[1])
    o_ref[...] = jnp.dot(h, w_ref[...], preferred_element_type=jnp.float32)

  return pl.pallas_call(
      body,
      grid=(_N // blk,),
      in_specs=[
          pl.BlockSpec((2, blk, _H1), lambda i: (0, i, 0)),
          pl.BlockSpec((_H1, 128), lambda i: (0, 0)),
      ],
      out_specs=pl.BlockSpec((blk, 128), lambda i: (i, 0)),
      out_shape=jax.ShapeDtypeStruct((_N, 128), jnp.float32),
  )(p, w2p)


def _add2(q):
  blk = 1000

  def body(q_ref, o_ref):
    o_ref[...] = (q_ref[0] + q_ref[1])[:, :_H2]

  return pl.pallas_call(
      body,
      grid=(_N // blk,),
      in_specs=[pl.BlockSpec((2, blk, 128), lambda i: (0, i, 0))],
      out_specs=pl.BlockSpec((blk, _H2), lambda i: (i, 0)),
      out_shape=jax.ShapeDtypeStruct((_N, _H2), jnp.float32),
  )(q)


def _zzt(z):
  blk = 400

  def body(zi_ref, zj_ref, o_ref):
    o_ref[...] = lax.dot_general(
        zi_ref[...], zj_ref[...], (((1,), (1,)), ((), ())),
        preferred_element_type=jnp.float32)

  return pl.pallas_call(
      body,
      grid=(_N // blk,),
      in_specs=[
          pl.BlockSpec((blk, _H2), lambda i: (i, 0)),
          pl.BlockSpec((_N, _H2), lambda i: (0, 0)),
      ],
      out_specs=pl.BlockSpec((blk, _N), lambda i: (i, 0)),
      out_shape=jax.ShapeDtypeStruct((_N, _N), jnp.float32),
  )(z, z)


def _make_seg_sum(h_feat, scale_feat):
  edges_per_worker = _E // _NW
  n_chunks = edges_per_worker // _CHUNK
  rows_per_tile = _NP // _NS
  mesh = plsc.VectorSubcoreMesh(
      core_axis_name="c", subcore_axis_name="s",
      num_cores=_NC, num_subcores=_NS)

  @functools.partial(
      pl.kernel,
      out_type=jax.ShapeDtypeStruct((_NC, _NP, h_feat), jnp.float32),
      mesh=mesh,
      scratch_types=[
          pltpu.VMEM((edges_per_worker,), jnp.int32),
          pltpu.VMEM((n_chunks, _CHUNK), jnp.int32),
          pltpu.VMEM((1, _CHUNK), jnp.float32),
          pltpu.VMEM((1, _CHUNK), jnp.float32),
          pltpu.VMEM((_CHUNK, h_feat), jnp.float32),
          pltpu.VMEM((_CHUNK, h_feat), jnp.float32),
          pltpu.VMEM_SHARED((_NP, h_feat), jnp.float32),
          pltpu.SemaphoreType.DMA,
          pltpu.SemaphoreType.DMA,
          pltpu.SemaphoreType.DMA,
          pltpu.SemaphoreType.DMA,
      ],
  )
  def seg(h_hbm, src_hbm, dst_hbm, w_hbm, zeros_hbm, out_hbm,
          src_v, dst_v, w_a, w_b, rows_a, rows_b, acc_sh,
          sem_a, sem_b, sem_wa, sem_wb):
    cid = lax.axis_index("c")
    sid = lax.axis_index("s")
    wid = cid * _NS + sid

    pltpu.sync_copy(src_hbm.at[wid], src_v)
    pltpu.sync_copy(dst_hbm.at[wid], dst_v)

    r0 = sid * rows_per_tile
    pltpu.sync_copy(zeros_hbm.at[pl.ds(r0, rows_per_tile)],
                    acc_sh.at[pl.ds(r0, rows_per_tile)])
    plsc.subcore_barrier()

    def issue_gather(ci, buf, sem):
      pltpu.async_copy(h_hbm.at[src_v.at[pl.ds(ci * _CHUNK, _CHUNK)]],
                       buf, sem)

    def wait_gather(ci, buf, sem):
      pltpu.make_async_copy(h_hbm.at[src_v.at[pl.ds(ci * _CHUNK, _CHUNK)]],
                            buf, sem).wait()

    def issue_w(ci, wbuf, sem):
      pltpu.async_copy(w_hbm.at[wid, pl.ds(ci, 1)], wbuf, sem)

    def wait_w(ci, wbuf, sem):
      pltpu.make_async_copy(w_hbm.at[wid, pl.ds(ci, 1)], wbuf, sem).wait()

    def scale(buf, wbuf):
      def group_body(g, carry2):
        wvec = wbuf[0, pl.ds(g * 16, 16)]
        for l in range(16):
          r = g * 16 + l
          wr = wvec[l]
          for j in range(scale_feat // 16):
            sl = pl.ds(j * 16, 16)
            buf[r, sl] = buf[r, sl] * wr
        return carry2

      lax.fori_loop(0, _CHUNK // 16, group_body, 0, unroll=False)

    def scatter(buf, ci):
      pltpu.sync_copy(buf, acc_sh.at[dst_v.at[ci]], add=True)

    issue_gather(0, rows_a, sem_a)
    issue_w(0, w_a, sem_wa)

    def pair_body(k, carry):
      c0 = 2 * k
      issue_gather(c0 + 1, rows_b, sem_b)
      issue_w(c0 + 1, w_b, sem_wb)
      wait_gather(c0, rows_a, sem_a)
      wait_w(c0, w_a, sem_wa)
      scale(rows_a, w_a)
      scatter(rows_a, c0)
      issue_gather(c0 + 2, rows_a, sem_a)
      issue_w(c0 + 2, w_a, sem_wa)
      wait_gather(c0 + 1, rows_b, sem_b)
      wait_w(c0 + 1, w_b, sem_wb)
      scale(rows_b, w_b)
      scatter(rows_b, c0 + 1)
      return carry

    lax.fori_loop(0, (n_chunks - 1) // 2, pair_body, 0, unroll=False)
    last = n_chunks - 1
    wait_gather(last, rows_a, sem_a)
    wait_w(last, w_a, sem_wa)
    scale(rows_a, w_a)
    scatter(rows_a, last)

    plsc.subcore_barrier()

    pltpu.sync_copy(acc_sh.at[pl.ds(r0, rows_per_tile)],
                    out_hbm.at[cid, pl.ds(r0, rows_per_tile)])

  return seg


_seg_sum_h1 = _make_seg_sum(_H1, _H1)
_seg_sum_h2 = _make_seg_sum(128, _H2)


def kernel(x, edge_index, edge_weight, W1, W2):
  nchk = _E // _NW // _CHUNK
  src = edge_index[0].astype(jnp.int32).reshape(_NW, _E // _NW)
  dst = edge_index[1].astype(jnp.int32).reshape(_NW, nchk, _CHUNK)
  w = edge_weight.astype(jnp.float32).reshape(_NW, nchk, _CHUNK)
  zeros1 = jnp.zeros((_NP, _H1), jnp.float32)
  zeros2 = jnp.zeros((_NP, 128), jnp.float32)

  h = _mm_xw1(x, W1)
  p = _seg_sum_h1(h, src, dst, w, zeros1)
  w2p = jnp.pad(W2, ((0, 0), (0, 128 - _H2)))
  h2 = _relu_add_mm(p, w2p)
  q = _seg_sum_h2(h2, src, dst, w, zeros2)
  z = _add2(q)
  adj = _zzt(z)
  return (adj, z)


# speedup vs baseline: 8.4567x; 1.0873x over previous
import functools

import jax
import jax.numpy as jnp
from jax import lax
from jax.experimental import pallas as pl
from jax.experimental.pallas import tpu as pltpu
from jax.experimental.pallas import tpu_sc as plsc

_N = 10000
_E = 320000
_D = 256
_H1 = 128
_H2 = 64

_NP = 10240
_NC = 2
_NS = 16
_NW = _NC * _NS
_CHUNK = 80


def _mm_xw1(x, w1):
  blk = 1000

  def body(x_ref, w_ref, o_ref):
    o_ref[...] = jnp.dot(x_ref[...], w_ref[...],
                         preferred_element_type=jnp.float32)

  return pl.pallas_call(
      body,
      grid=(_N // blk,),
      in_specs=[
          pl.BlockSpec((blk, _D), lambda i: (i, 0)),
          pl.BlockSpec((_D, _H1), lambda i: (0, 0)),
      ],
      out_specs=pl.BlockSpec((blk, _H1), lambda i: (i, 0)),
      out_shape=jax.ShapeDtypeStruct((_N, _H1), jnp.float32),
  )(x, w1)


def _relu_add_mm(p, w2p):
  blk = 1000

  def body(p_ref, w_ref, o_ref):
    h = jax.nn.relu(p_ref[0] + p_ref[1])
    o_ref[...] = jnp.dot(h, w_ref[...], preferred_element_type=jnp.float32)

  return pl.pallas_call(
      body,
      grid=(_N // blk,),
      in_specs=[
          pl.BlockSpec((2, blk, _H1), lambda i: (0, i, 0)),
          pl.BlockSpec((_H1, 128), lambda i: (0, 0)),
      ],
      out_specs=pl.BlockSpec((blk, 128), lambda i: (i, 0)),
      out_shape=jax.ShapeDtypeStruct((_N, 128), jnp.float32),
  )(p, w2p)


def _add2(q):
  blk = 1000

  def body(q_ref, o_ref):
    o_ref[...] = (q_ref[0] + q_ref[1])[:, :_H2]

  return pl.pallas_call(
      body,
      grid=(_N // blk,),
      in_specs=[pl.BlockSpec((2, blk, 128), lambda i: (0, i, 0))],
      out_specs=pl.BlockSpec((blk, _H2), lambda i: (i, 0)),
      out_shape=jax.ShapeDtypeStruct((_N, _H2), jnp.float32),
  )(q)


def _zzt(z):
  blk = 400

  def body(zi_ref, zj_ref, o_ref):
    o_ref[...] = lax.dot_general(
        zi_ref[...], zj_ref[...], (((1,), (1,)), ((), ())),
        preferred_element_type=jnp.float32)

  return pl.pallas_call(
      body,
      grid=(_N // blk,),
      in_specs=[
          pl.BlockSpec((blk, _H2), lambda i: (i, 0)),
          pl.BlockSpec((_N, _H2), lambda i: (0, 0)),
      ],
      out_specs=pl.BlockSpec((blk, _N), lambda i: (i, 0)),
      out_shape=jax.ShapeDtypeStruct((_N, _N), jnp.float32),
  )(z, z)


def _make_seg_sum(h_feat, scale_feat):
  edges_per_worker = _E // _NW
  n_chunks = edges_per_worker // _CHUNK
  rows_per_tile = _NP // _NS
  mesh = plsc.VectorSubcoreMesh(
      core_axis_name="c", subcore_axis_name="s",
      num_cores=_NC, num_subcores=_NS)

  @functools.partial(
      pl.kernel,
      out_type=jax.ShapeDtypeStruct((_NC, _NP, h_feat), jnp.float32),
      mesh=mesh,
      scratch_types=[
          pltpu.VMEM((edges_per_worker,), jnp.int32),
          [pltpu.VMEM((1, _CHUNK), jnp.int32) for _ in range(3)],
          [pltpu.VMEM((1, _CHUNK), jnp.float32) for _ in range(3)],
          [pltpu.VMEM((_CHUNK, h_feat), jnp.float32) for _ in range(3)],
          pltpu.VMEM_SHARED((_NP, h_feat), jnp.float32),
          [pltpu.SemaphoreType.DMA for _ in range(3)],
          [pltpu.SemaphoreType.DMA for _ in range(3)],
          [pltpu.SemaphoreType.DMA for _ in range(3)],
          [pltpu.SemaphoreType.DMA for _ in range(3)],
      ],
  )
  def seg(h_hbm, src_hbm, dst_hbm, w_hbm, zeros_hbm, out_hbm,
          src_v, dst_bufs, w_bufs, row_bufs, acc_sh,
          g_sems, d_sems, w_sems, s_sems):
    cid = lax.axis_index("c")
    sid = lax.axis_index("s")
    wid = cid * _NS + sid

    pltpu.sync_copy(src_hbm.at[wid], src_v)

    r0 = sid * rows_per_tile
    pltpu.sync_copy(zeros_hbm.at[pl.ds(r0, rows_per_tile)],
                    acc_sh.at[pl.ds(r0, rows_per_tile)])
    plsc.subcore_barrier()

    def gather_desc(ci, b):
      return pltpu.make_async_copy(
          h_hbm.at[src_v.at[pl.ds(ci * _CHUNK, _CHUNK)]],
          row_bufs[b], g_sems[b])

    def dw_desc(ci, b):
      return pltpu.make_async_copy(
          dst_hbm.at[wid, pl.ds(ci, 1)], dst_bufs[b], d_sems[b])

    def w_desc(ci, b):
      return pltpu.make_async_copy(
          w_hbm.at[wid, pl.ds(ci, 1)], w_bufs[b], w_sems[b])

    def issue_both(ci, b):
      pltpu.async_copy(h_hbm.at[src_v.at[pl.ds(ci * _CHUNK, _CHUNK)]],
                       row_bufs[b], g_sems[b])
      pltpu.async_copy(dst_hbm.at[wid, pl.ds(ci, 1)], dst_bufs[b], d_sems[b])
      pltpu.async_copy(w_hbm.at[wid, pl.ds(ci, 1)], w_bufs[b], w_sems[b])

    def scatter_desc(b):
      return pltpu.make_async_copy(
          row_bufs[b], acc_sh.at[dst_bufs[b].at[0]], s_sems[b])

    def issue_scatter(b):
      pltpu.async_copy(row_bufs[b], acc_sh.at[dst_bufs[b].at[0]],
                       s_sems[b], add=True)

    def scale(b):
      buf = row_bufs[b]
      wb = w_bufs[b]

      def group_body(g, carry2):
        wvec = wb[0, pl.ds(g * 16, 16)]
        for l in range(16):
          r = g * 16 + l
          wr = wvec[l]
          for j in range(scale_feat // 16):
            sl = pl.ds(j * 16, 16)
            buf[r, sl] = buf[r, sl] * wr
        return carry2

      lax.fori_loop(0, _CHUNK // 16, group_body, 0, unroll=False)

    def slot(c, b, first):
      gather_desc(c, b).wait()
      dw_desc(c, b).wait()
      w_desc(c, b).wait()
      scale(b)
      zb = (b + 2) % 3
      if not first:
        scatter_desc(zb).wait()

      @pl.when(c + 2 < n_chunks)
      def _():
        issue_both(c + 2, zb)

      issue_scatter(b)

    issue_both(0, 0)
    issue_both(1, 1)
    slot(0, 0, first=True)

    def triple_body(k, carry):
      c = 3 * k + 1
      slot(c, 1, first=False)
      slot(c + 1, 2, first=False)
      slot(c + 2, 0, first=False)
      return carry

    lax.fori_loop(0, (n_chunks - 2) // 3, triple_body, 0, unroll=False)

    slot(n_chunks - 1, (n_chunks - 1) % 3, first=False)
    scatter_desc((n_chunks - 1) % 3).wait()

    plsc.subcore_barrier()

    pltpu.sync_copy(acc_sh.at[pl.ds(r0, rows_per_tile)],
                    out_hbm.at[cid, pl.ds(r0, rows_per_tile)])

  return seg


_seg_sum_h1 = _make_seg_sum(_H1, _H1)
_seg_sum_h2 = _make_seg_sum(128, _H2)


def kernel(x, edge_index, edge_weight, W1, W2):
  nchk = _E // _NW // _CHUNK
  src = edge_index[0].astype(jnp.int32).reshape(_NW, _E // _NW)
  dst = edge_index[1].astype(jnp.int32).reshape(_NW, nchk, _CHUNK)
  w = edge_weight.astype(jnp.float32).reshape(_NW, nchk, _CHUNK)
  zeros1 = jnp.zeros((_NP, _H1), jnp.float32)
  zeros2 = jnp.zeros((_NP, 128), jnp.float32)

  h = _mm_xw1(x, W1)
  p = _seg_sum_h1(h, src, dst, w, zeros1)
  w2p = jnp.pad(W2, ((0, 0), (0, 128 - _H2)))
  h2 = _relu_add_mm(p, w2p)
  q = _seg_sum_h2(h2, src, dst, w, zeros2)
  z = _add2(q)
  adj = _zzt(z)
  return (adj, z)


# speedup vs baseline: 8.8785x; 1.0499x over previous
import functools

import jax
import jax.numpy as jnp
from jax import lax
from jax.experimental import pallas as pl
from jax.experimental.pallas import tpu as pltpu
from jax.experimental.pallas import tpu_sc as plsc

_N = 10000
_E = 320000
_D = 256
_H1 = 128
_H2 = 64

_NP = 10240
_NC = 2
_NS = 16
_NW = _NC * _NS
_CHUNK = 80


def _mm_xw1(x, w1):
  blk = 1000

  def body(x_ref, w_ref, o_ref):
    o_ref[...] = jnp.dot(x_ref[...].astype(jnp.bfloat16),
                         w_ref[...].astype(jnp.bfloat16),
                         preferred_element_type=jnp.float32)

  return pl.pallas_call(
      body,
      grid=(_N // blk,),
      in_specs=[
          pl.BlockSpec((blk, _D), lambda i: (i, 0)),
          pl.BlockSpec((_D, _H1), lambda i: (0, 0)),
      ],
      out_specs=pl.BlockSpec((blk, _H1), lambda i: (i, 0)),
      out_shape=jax.ShapeDtypeStruct((_N, _H1), jnp.float32),
  )(x, w1)


def _relu_add_mm(p, w2):
  blk = 1000

  def body(p_ref, w_ref, o_ref):
    h = jax.nn.relu(p_ref[0] + p_ref[1])
    o_ref[...] = jnp.dot(h.astype(jnp.bfloat16),
                         w_ref[...].astype(jnp.bfloat16),
                         preferred_element_type=jnp.float32)

  return pl.pallas_call(
      body,
      grid=(_N // blk,),
      in_specs=[
          pl.BlockSpec((2, blk, _H1), lambda i: (0, i, 0)),
          pl.BlockSpec((_H1, _H2), lambda i: (0, 0)),
      ],
      out_specs=pl.BlockSpec((blk, _H2), lambda i: (i, 0)),
      out_shape=jax.ShapeDtypeStruct((_N, _H2), jnp.float32),
  )(p, w2)


def _zzt_fused(q):
  blk = 400

  def body(qi_ref, qall_ref, adj_ref, z_ref):
    zi = qi_ref[0] + qi_ref[1]
    zall = qall_ref[0] + qall_ref[1]
    z_ref[...] = zi
    adj_ref[...] = lax.dot_general(
        zi.astype(jnp.bfloat16), zall.astype(jnp.bfloat16),
        (((1,), (1,)), ((), ())),
        preferred_element_type=jnp.float32)

  return pl.pallas_call(
      body,
      grid=(_N // blk,),
      in_specs=[
          pl.BlockSpec((2, blk, _H2), lambda i: (0, i, 0)),
          pl.BlockSpec((2, _N, _H2), lambda i: (0, 0, 0)),
      ],
      out_specs=[
          pl.BlockSpec((blk, _N), lambda i: (i, 0)),
          pl.BlockSpec((blk, _H2), lambda i: (i, 0)),
      ],
      out_shape=[
          jax.ShapeDtypeStruct((_N, _N), jnp.float32),
          jax.ShapeDtypeStruct((_N, _H2), jnp.float32),
      ],
  )(q, q)


def _make_seg_sum(h_feat, scale_feat):
  edges_per_worker = _E // _NW
  n_chunks = edges_per_worker // _CHUNK
  rows_per_tile = _NP // _NS
  mesh = plsc.VectorSubcoreMesh(
      core_axis_name="c", subcore_axis_name="s",
      num_cores=_NC, num_subcores=_NS)

  @functools.partial(
      pl.kernel,
      out_type=jax.ShapeDtypeStruct((_NC, _NP, h_feat), jnp.float32),
      mesh=mesh,
      compiler_params=pltpu.CompilerParams(use_tc_tiling_on_sc=False),
      scratch_types=[
          pltpu.VMEM((edges_per_worker,), jnp.int32),
          [pltpu.VMEM((1, _CHUNK), jnp.int32) for _ in range(3)],
          [pltpu.VMEM((1, _CHUNK), jnp.float32) for _ in range(3)],
          [pltpu.VMEM((_CHUNK, h_feat), jnp.float32) for _ in range(3)],
          pltpu.VMEM_SHARED((_NP, h_feat), jnp.float32),
          [pltpu.SemaphoreType.DMA for _ in range(3)],
          [pltpu.SemaphoreType.DMA for _ in range(3)],
          [pltpu.SemaphoreType.DMA for _ in range(3)],
          [pltpu.SemaphoreType.DMA for _ in range(3)],
          pltpu.SemaphoreType.DMA,
      ],
  )
  def seg(h_hbm, src_hbm, dst_hbm, w_hbm, zeros_hbm, out_hbm,
          src_v, dst_bufs, w_bufs, row_bufs, acc_sh,
          g_sems, d_sems, w_sems, s_sems, z_sem):
    cid = lax.axis_index("c")
    sid = lax.axis_index("s")
    wid = cid * _NS + sid

    r0 = sid * rows_per_tile
    zero_cp = pltpu.async_copy(zeros_hbm.at[pl.ds(r0, rows_per_tile)],
                               acc_sh.at[pl.ds(r0, rows_per_tile)], z_sem)
    pltpu.sync_copy(src_hbm.at[wid], src_v)
    zero_cp.wait()
    plsc.subcore_barrier()

    def gather_desc(ci, b):
      return pltpu.make_async_copy(
          h_hbm.at[src_v.at[pl.ds(ci * _CHUNK, _CHUNK)]],
          row_bufs[b], g_sems[b])

    def dw_desc(ci, b):
      return pltpu.make_async_copy(
          dst_hbm.at[wid, pl.ds(ci, 1)], dst_bufs[b], d_sems[b])

    def w_desc(ci, b):
      return pltpu.make_async_copy(
          w_hbm.at[wid, pl.ds(ci, 1)], w_bufs[b], w_sems[b])

    def issue_both(ci, b):
      pltpu.async_copy(h_hbm.at[src_v.at[pl.ds(ci * _CHUNK, _CHUNK)]],
                       row_bufs[b], g_sems[b])
      pltpu.async_copy(dst_hbm.at[wid, pl.ds(ci, 1)], dst_bufs[b], d_sems[b])
      pltpu.async_copy(w_hbm.at[wid, pl.ds(ci, 1)], w_bufs[b], w_sems[b])

    def scatter_desc(b):
      return pltpu.make_async_copy(
          row_bufs[b], acc_sh.at[dst_bufs[b].at[0]], s_sems[b])

    def issue_scatter(b):
      pltpu.async_copy(row_bufs[b], acc_sh.at[dst_bufs[b].at[0]],
                       s_sems[b], add=True)

    def scale(b):
      buf = row_bufs[b]
      wb = w_bufs[b]

      def group_body(g, carry2):
        wvec = wb[0, pl.ds(g * 16, 16)]
        for l in range(16):
          r = g * 16 + l
          wr = wvec[l]
          for j in range(scale_feat // 16):
            sl = pl.ds(j * 16, 16)
            buf[r, sl] = buf[r, sl] * wr
        return carry2

      lax.fori_loop(0, _CHUNK // 16, group_body, 0, unroll=False)

    def slot(c, b, first):
      gather_desc(c, b).wait()
      dw_desc(c, b).wait()
      w_desc(c, b).wait()
      scale(b)
      zb = (b + 2) % 3
      if not first:
        scatter_desc(zb).wait()

      @pl.when(c + 2 < n_chunks)
      def _():
        issue_both(c + 2, zb)

      issue_scatter(b)

    issue_both(0, 0)
    issue_both(1, 1)
    slot(0, 0, first=True)

    def triple_body(k, carry):
      c = 3 * k + 1
      slot(c, 1, first=False)
      slot(c + 1, 2, first=False)
      slot(c + 2, 0, first=False)
      return carry

    lax.fori_loop(0, (n_chunks - 2) // 3, triple_body, 0, unroll=False)

    slot(n_chunks - 1, (n_chunks - 1) % 3, first=False)
    scatter_desc((n_chunks - 1) % 3).wait()

    plsc.subcore_barrier()

    pltpu.sync_copy(acc_sh.at[pl.ds(r0, rows_per_tile)],
                    out_hbm.at[cid, pl.ds(r0, rows_per_tile)])

  return seg


_seg_sum_h1 = _make_seg_sum(_H1, _H1)
_seg_sum_h2 = _make_seg_sum(_H2, _H2)


def kernel(x, edge_index, edge_weight, W1, W2):
  epw = _E // _NW
  nchk = epw // _CHUNK
  src = edge_index[0].astype(jnp.int32).reshape(_NW, epw)
  dst = edge_index[1].astype(jnp.int32).reshape(_NW, nchk, _CHUNK)
  w = edge_weight.astype(jnp.float32).reshape(_NW, nchk, _CHUNK)
  zeros1 = jnp.zeros((_NP, _H1), jnp.float32)
  zeros2 = jnp.zeros((_NP, _H2), jnp.float32)

  h = _mm_xw1(x, W1)
  p = _seg_sum_h1(h, src, dst, w, zeros1)
  h2 = _relu_add_mm(p, W2)
  q = _seg_sum_h2(h2, src, dst, w, zeros2)
  adj, z = _zzt_fused(q)
  return (adj, z)
